# baseline (speedup 1.0000x reference)
import functools

import jax
import jax.numpy as jnp
from jax import lax
from jax.experimental import pallas as pl
from jax.experimental.pallas import tpu as pltpu
from jax.experimental.pallas import tpu_sc as plsc

B = 16384
D = 128
NC = 2
NS = 16
NW = NC * NS
BPW = B // NW
CHUNK = 32
NCHUNK = BPW // CHUNK
NSETS = 12
LANES = 16


def _gmf_body(ut_hbm, it_hbm, ui_hbm, ii_hbm, out_hbm,
              ui_v, ii_v, u_buf, i_buf, *sems):
    sem_g = sems[:NSETS]
    sem_o = sems[NSETS:2 * NSETS]
    sem_ix = sems[2 * NSETS]
    wid = lax.axis_index("s") * NC + lax.axis_index("c")
    base = wid * BPW

    cu = pltpu.async_copy(ui_hbm.at[wid], ui_v, sem_ix)
    ci = pltpu.async_copy(ii_hbm.at[wid], ii_v, sem_ix)
    cu.wait()
    ci.wait()

    def gathers(j, s):
        cu = pltpu.async_copy(ut_hbm.at[ui_v.at[j]], u_buf.at[s], sem_g[s])
        ci = pltpu.async_copy(it_hbm.at[ii_v.at[j]], i_buf.at[s], sem_g[s])
        return cu, ci

    pend_g = [gathers(k, k) for k in range(min(NSETS, NCHUNK))]
    pend_o = [None] * NSETS
    for j in range(NCHUNK):
        s = j % NSETS
        pend_g[s][0].wait()
        pend_g[s][1].wait()

        def row_body(r, carry):
            for g in range(D // LANES):
                sl = pl.ds(g * LANES, LANES)
                u_buf[s, r, sl] = u_buf[s, r, sl] * i_buf[s, r, sl]
            return carry

        lax.fori_loop(0, CHUNK, row_body, 0)
        pend_o[s] = pltpu.async_copy(
            u_buf.at[s], out_hbm.at[pl.ds(base + j * CHUNK, CHUNK)], sem_o[s])
        if j + NSETS < NCHUNK:
            pend_o[s].wait()
            pend_o[s] = None
            pend_g[s] = gathers(j + NSETS, s)
    for s in range(NSETS):
        if pend_o[s] is not None:
            pend_o[s].wait()


@functools.partial(jax.jit, static_argnames=())
def _gmf(user_table, item_table, user_indices, item_indices):
    mesh = plsc.VectorSubcoreMesh(core_axis_name="c", subcore_axis_name="s")
    call = pl.kernel(
        _gmf_body,
        mesh=mesh,
        out_type=jax.ShapeDtypeStruct((B, D), jnp.float32),
        scratch_types=[
            pltpu.VMEM((NCHUNK, CHUNK), jnp.int32),
            pltpu.VMEM((NCHUNK, CHUNK), jnp.int32),
            pltpu.VMEM((NSETS, CHUNK, D), jnp.float32),
            pltpu.VMEM((NSETS, CHUNK, D), jnp.float32),
        ] + [pltpu.SemaphoreType.DMA] * (2 * NSETS + 1),
    )
    return call(user_table, item_table, user_indices, item_indices)


def kernel(user_indices, item_indices, user_table, item_table):
    ui = user_indices.astype(jnp.int32).reshape(NW, NCHUNK, CHUNK)
    ii = item_indices.astype(jnp.int32).reshape(NW, NCHUNK, CHUNK)
    return _gmf(user_table, item_table, ui, ii)


# speedup vs baseline: 1.0117x; 1.0117x over previous
import functools

import jax
import jax.numpy as jnp
from jax import lax
from jax.experimental import pallas as pl
from jax.experimental.pallas import tpu as pltpu
from jax.experimental.pallas import tpu_sc as plsc

B = 16384
D = 128
NC = 2
NS = 16
NW = NC * NS
BPW = B // NW
CHUNK = 128
NCHUNK = BPW // CHUNK
NSETS = 3
LANES = 16


def _gmf_body(ut_hbm, it_hbm, ui_hbm, ii_hbm, out_hbm,
              ui_v, ii_v, u_buf, i_buf, *sems):
    sem_g = sems[:NSETS]
    sem_o = sems[NSETS:2 * NSETS]
    sem_ix = sems[2 * NSETS]
    wid = lax.axis_index("s") * NC + lax.axis_index("c")
    base = wid * BPW

    cu = pltpu.async_copy(ui_hbm.at[wid], ui_v, sem_ix)
    ci = pltpu.async_copy(ii_hbm.at[wid], ii_v, sem_ix)
    cu.wait()
    ci.wait()

    def gathers(j, s):
        cu = pltpu.async_copy(ut_hbm.at[ui_v.at[j]], u_buf.at[s], sem_g[s])
        ci = pltpu.async_copy(it_hbm.at[ii_v.at[j]], i_buf.at[s], sem_g[s])
        return cu, ci

    pend_g = [gathers(k, k) for k in range(min(NSETS, NCHUNK))]
    pend_o = [None] * NSETS
    for j in range(NCHUNK):
        s = j % NSETS
        pend_g[s][0].wait()
        pend_g[s][1].wait()

        def row_body(r, carry):
            for g in range(D // LANES):
                sl = pl.ds(g * LANES, LANES)
                u_buf[s, r, sl] = u_buf[s, r, sl] * i_buf[s, r, sl]
            return carry

        lax.fori_loop(0, CHUNK, row_body, 0)
        pend_o[s] = pltpu.async_copy(
            u_buf.at[s], out_hbm.at[pl.ds(base + j * CHUNK, CHUNK)], sem_o[s])
        if j + NSETS < NCHUNK:
            pend_o[s].wait()
            pend_o[s] = None
            pend_g[s] = gathers(j + NSETS, s)
    for s in range(NSETS):
        if pend_o[s] is not None:
            pend_o[s].wait()


@functools.partial(jax.jit, static_argnames=())
def _gmf(user_table, item_table, user_indices, item_indices):
    mesh = plsc.VectorSubcoreMesh(core_axis_name="c", subcore_axis_name="s")
    call = pl.kernel(
        _gmf_body,
        mesh=mesh,
        out_type=jax.ShapeDtypeStruct((B, D), jnp.float32),
        scratch_types=[
            pltpu.VMEM((NCHUNK, CHUNK), jnp.int32),
            pltpu.VMEM((NCHUNK, CHUNK), jnp.int32),
            pltpu.VMEM((NSETS, CHUNK, D), jnp.float32),
            pltpu.VMEM((NSETS, CHUNK, D), jnp.float32),
        ] + [pltpu.SemaphoreType.DMA] * (2 * NSETS + 1),
    )
    return call(user_table, item_table, user_indices, item_indices)


def kernel(user_indices, item_indices, user_table, item_table):
    ui = user_indices.astype(jnp.int32).reshape(NW, NCHUNK, CHUNK)
    ii = item_indices.astype(jnp.int32).reshape(NW, NCHUNK, CHUNK)
    return _gmf(user_table, item_table, ui, ii)


# speedup vs baseline: 1.0333x; 1.0214x over previous
import functools

import jax
import jax.numpy as jnp
from jax import lax
from jax.experimental import pallas as pl
from jax.experimental.pallas import tpu as pltpu
from jax.experimental.pallas import tpu_sc as plsc

B = 16384
D = 128
NC = 2
NS = 16
NW = NC * NS
BPW = B // NW
CHUNK = 64
NCHUNK = BPW // CHUNK
NSETS = 7
LANES = 16


def _gmf_body(ut_hbm, it_hbm, ui_hbm, ii_hbm, out_hbm,
              ui_v, ii_v, u_buf, i_buf, *sems):
    sem_g = sems[:NSETS]
    sem_o = sems[NSETS:2 * NSETS]
    sem_ix = sems[2 * NSETS]
    wid = lax.axis_index("s") * NC + lax.axis_index("c")
    base = wid * BPW

    cu = pltpu.async_copy(ui_hbm.at[wid], ui_v, sem_ix)
    ci = pltpu.async_copy(ii_hbm.at[wid], ii_v, sem_ix)
    cu.wait()
    ci.wait()

    def gathers(j, s):
        cu = pltpu.async_copy(ut_hbm.at[ui_v.at[j]], u_buf.at[s], sem_g[s])
        ci = pltpu.async_copy(it_hbm.at[ii_v.at[j]], i_buf.at[s], sem_g[s])
        return cu, ci

    pend_g = [gathers(k, k) for k in range(min(NSETS, NCHUNK))]
    pend_o = [None] * NSETS
    for j in range(NCHUNK):
        s = j % NSETS
        pend_g[s][0].wait()
        pend_g[s][1].wait()

        def row_body(r, carry):
            for g in range(D // LANES):
                sl = pl.ds(g * LANES, LANES)
                u_buf[s, r, sl] = u_buf[s, r, sl] * i_buf[s, r, sl]
            return carry

        lax.fori_loop(0, CHUNK, row_body, 0)
        pend_o[s] = pltpu.async_copy(
            u_buf.at[s], out_hbm.at[pl.ds(base + j * CHUNK, CHUNK)], sem_o[s])
        if j + NSETS < NCHUNK:
            pend_o[s].wait()
            pend_o[s] = None
            pend_g[s] = gathers(j + NSETS, s)
    for s in range(NSETS):
        if pend_o[s] is not None:
            pend_o[s].wait()


@functools.partial(jax.jit, static_argnames=())
def _gmf(user_table, item_table, user_indices, item_indices):
    mesh = plsc.VectorSubcoreMesh(core_axis_name="c", subcore_axis_name="s")
    call = pl.kernel(
        _gmf_body,
        mesh=mesh,
        out_type=jax.ShapeDtypeStruct((B, D), jnp.float32),
        scratch_types=[
            pltpu.VMEM((NCHUNK, CHUNK), jnp.int32),
            pltpu.VMEM((NCHUNK, CHUNK), jnp.int32),
            pltpu.VMEM((NSETS, CHUNK, D), jnp.float32),
            pltpu.VMEM((NSETS, CHUNK, D), jnp.float32),
        ] + [pltpu.SemaphoreType.DMA] * (2 * NSETS + 1),
    )
    return call(user_table, item_table, user_indices, item_indices)


def kernel(user_indices, item_indices, user_table, item_table):
    ui = user_indices.astype(jnp.int32).reshape(NW, NCHUNK, CHUNK)
    ii = item_indices.astype(jnp.int32).reshape(NW, NCHUNK, CHUNK)
    return _gmf(user_table, item_table, ui, ii)


# speedup vs baseline: 1.0469x; 1.0131x over previous
import functools

import jax
import jax.numpy as jnp
from jax import lax
from jax.experimental import pallas as pl
from jax.experimental.pallas import tpu as pltpu
from jax.experimental.pallas import tpu_sc as plsc

B = 16384
D = 128
NC = 2
NS = 16
NW = NC * NS
BPW = B // NW
CHUNK = 64
NCHUNK = BPW // CHUNK
NSETS = 7
LANES = 16


def _gmf_body(ut_hbm, it_hbm, ui_hbm, ii_hbm, out_hbm,
              ui_v, ii_v, u_buf, i_buf, *sems):
    sem_g = sems[:NSETS]
    sem_o = sems[NSETS:2 * NSETS]
    sem_ix = sems[2 * NSETS]
    wid = lax.axis_index("s") * NC + lax.axis_index("c")
    base = wid * BPW

    idx_copies = []
    for j in range(NCHUNK):
        idx_copies.append(pltpu.async_copy(
            ui_hbm.at[pl.ds(base + j * CHUNK, CHUNK)], ui_v.at[j], sem_ix))
        idx_copies.append(pltpu.async_copy(
            ii_hbm.at[pl.ds(base + j * CHUNK, CHUNK)], ii_v.at[j], sem_ix))
    for c in idx_copies:
        c.wait()

    def gathers(j, s):
        cu = pltpu.async_copy(ut_hbm.at[ui_v.at[j]], u_buf.at[s], sem_g[s])
        ci = pltpu.async_copy(it_hbm.at[ii_v.at[j]], i_buf.at[s], sem_g[s])
        return cu, ci

    pend_g = [gathers(k, k) for k in range(min(NSETS, NCHUNK))]
    pend_o = [None] * NSETS
    for j in range(NCHUNK):
        s = j % NSETS
        pend_g[s][0].wait()
        pend_g[s][1].wait()

        def row_body(r, carry):
            for g in range(D // LANES):
                sl = pl.ds(g * LANES, LANES)
                u_buf[s, r, sl] = u_buf[s, r, sl] * i_buf[s, r, sl]
            return carry

        lax.fori_loop(0, CHUNK, row_body, 0)
        pend_o[s] = pltpu.async_copy(
            u_buf.at[s], out_hbm.at[pl.ds(base + j * CHUNK, CHUNK)], sem_o[s])
        if j + NSETS < NCHUNK:
            pend_o[s].wait()
            pend_o[s] = None
            pend_g[s] = gathers(j + NSETS, s)
    for s in range(NSETS):
        if pend_o[s] is not None:
            pend_o[s].wait()


@functools.partial(jax.jit, static_argnames=())
def _gmf(user_table, item_table, user_indices, item_indices):
    mesh = plsc.VectorSubcoreMesh(core_axis_name="c", subcore_axis_name="s")
    call = pl.kernel(
        _gmf_body,
        mesh=mesh,
        out_type=jax.ShapeDtypeStruct((B, D), jnp.float32),
        scratch_types=[
            pltpu.VMEM((NCHUNK, CHUNK), jnp.int32),
            pltpu.VMEM((NCHUNK, CHUNK), jnp.int32),
            pltpu.VMEM((NSETS, CHUNK, D), jnp.float32),
            pltpu.VMEM((NSETS, CHUNK, D), jnp.float32),
        ] + [pltpu.SemaphoreType.DMA] * (2 * NSETS + 1),
    )
    return call(user_table, item_table, user_indices, item_indices)


def kernel(user_indices, item_indices, user_table, item_table):
    return _gmf(user_table, item_table,
                user_indices.astype(jnp.int32), item_indices.astype(jnp.int32))


# speedup vs baseline: 1.0518x; 1.0047x over previous
import functools

import jax
import jax.numpy as jnp
from jax import lax
from jax.experimental import pallas as pl
from jax.experimental.pallas import tpu as pltpu
from jax.experimental.pallas import tpu_sc as plsc

B = 16384
D = 128
NC = 2
NS = 16
NW = NC * NS
BPW = B // NW
CHUNK = 64
NCHUNK = BPW // CHUNK
NSETS = 7
LANES = 16


def _gmf_body(ut_hbm, it_hbm, ui_hbm, ii_hbm, out_hbm,
              ui_v, ii_v, u_buf, i_buf, *sems):
    sem_g = sems[:NSETS]
    sem_o = sems[NSETS:2 * NSETS]
    sem_ix = sems[2 * NSETS]
    wid = lax.axis_index("s") * NC + lax.axis_index("c")
    base = wid * BPW

    cu = pltpu.async_copy(ui_hbm.at[pl.ds(base, BPW)], ui_v, sem_ix)
    ci = pltpu.async_copy(ii_hbm.at[pl.ds(base, BPW)], ii_v, sem_ix)
    cu.wait()
    ci.wait()

    def gathers(j, s):
        isl = pl.ds(j * CHUNK, CHUNK)
        cu = pltpu.async_copy(ut_hbm.at[ui_v.at[isl]], u_buf.at[s], sem_g[s])
        ci = pltpu.async_copy(it_hbm.at[ii_v.at[isl]], i_buf.at[s], sem_g[s])
        return cu, ci

    pend_g = [gathers(k, k) for k in range(min(NSETS, NCHUNK))]
    pend_o = [None] * NSETS
    for j in range(NCHUNK):
        s = j % NSETS
        pend_g[s][0].wait()
        pend_g[s][1].wait()

        def row_body(r, carry):
            for g in range(D // LANES):
                sl = pl.ds(g * LANES, LANES)
                u_buf[s, r, sl] = u_buf[s, r, sl] * i_buf[s, r, sl]
            return carry

        lax.fori_loop(0, CHUNK, row_body, 0)
        pend_o[s] = pltpu.async_copy(
            u_buf.at[s], out_hbm.at[pl.ds(base + j * CHUNK, CHUNK)], sem_o[s])
        if j + NSETS < NCHUNK:
            pend_o[s].wait()
            pend_o[s] = None
            pend_g[s] = gathers(j + NSETS, s)
    for s in range(NSETS):
        if pend_o[s] is not None:
            pend_o[s].wait()


@functools.partial(jax.jit, static_argnames=())
def _gmf(user_table, item_table, user_indices, item_indices):
    mesh = plsc.VectorSubcoreMesh(core_axis_name="c", subcore_axis_name="s")
    call = pl.kernel(
        _gmf_body,
        mesh=mesh,
        out_type=jax.ShapeDtypeStruct((B, D), jnp.float32),
        scratch_types=[
            pltpu.VMEM((BPW,), jnp.int32),
            pltpu.VMEM((BPW,), jnp.int32),
            pltpu.VMEM((NSETS, CHUNK, D), jnp.float32),
            pltpu.VMEM((NSETS, CHUNK, D), jnp.float32),
        ] + [pltpu.SemaphoreType.DMA] * (2 * NSETS + 1),
    )
    return call(user_table, item_table, user_indices, item_indices)


def kernel(user_indices, item_indices, user_table, item_table):
    return _gmf(user_table, item_table,
                user_indices.astype(jnp.int32), item_indices.astype(jnp.int32))
